# baseline (speedup 1.0000x reference)
import functools

import jax
import jax.numpy as jnp
from jax import lax
from jax.experimental import pallas as pl
from jax.experimental.pallas import tpu as pltpu
from jax.experimental.pallas import tpu_sc as plsc

_NC = 2
_NS = 16
_LANES = 16
_NW = _NC * _NS
_CB = 48
_GRP = 18
_NBUF = 3


def _sc_segsum(h, src_g, dst_g, ew_g, zeros_nd):
    n, d = h.shape
    np_ = zeros_nd.shape[0]
    nwg, grp, cb = src_g.shape
    n_groups = nwg // _NW
    k_chunks = n_groups * grp
    rows_per_sub = np_ // _NS
    nj = d // _LANES

    mesh = plsc.VectorSubcoreMesh(
        core_axis_name="c", subcore_axis_name="s",
        num_cores=_NC, num_subcores=_NS)

    @functools.partial(
        pl.kernel, mesh=mesh,
        out_type=jax.ShapeDtypeStruct((_NC * np_, d), jnp.float32),
        scratch_types=[
            pltpu.VMEM((_GRP, cb), jnp.int32),
            pltpu.VMEM((_GRP, cb), jnp.int32),
            pltpu.VMEM((_GRP * cb,), jnp.float32),
            pltpu.VMEM((_NBUF * cb, d), jnp.float32),
            pltpu.VMEM_SHARED((np_, d), jnp.float32),
            pltpu.SemaphoreType.DMA,
            pltpu.SemaphoreType.DMA,
        ])
    def seg_kernel(h_hbm, src_hbm, dst_hbm, ew_hbm, z_hbm, out_hbm,
                   src_v, dst_v, ew_v, rows_v, acc_sh, gsem, ssem):
        c = lax.axis_index("c")
        s = lax.axis_index("s")
        w = s * _NC + c
        pltpu.sync_copy(z_hbm.at[pl.ds(s * rows_per_sub, rows_per_sub)],
                        acc_sh.at[pl.ds(s * rows_per_sub, rows_per_sub)])
        plsc.subcore_barrier()

        def rows_at(kk):
            boff = lax.rem(kk, _NBUF) * cb if not isinstance(kk, int) \
                else (kk % _NBUF) * cb
            return rows_v.at[pl.ds(boff, cb)]

        def issue_gather(kk):
            pltpu.async_copy(h_hbm.at[src_v.at[kk]], rows_at(kk), gsem)

        def wait_gather(kk):
            pltpu.make_async_copy(h_hbm.at[src_v.at[kk]], rows_at(kk),
                                  gsem).wait()

        def issue_scatter(kk):
            pltpu.async_copy(rows_at(kk), acc_sh.at[dst_v.at[kk]], ssem,
                             add=True)

        def wait_scatter(kk):
            pltpu.make_async_copy(rows_at(kk), acc_sh.at[dst_v.at[kk]],
                                  ssem).wait()

        def scale(kk):
            boff = lax.rem(kk, _NBUF) * cb if not isinstance(kk, int) \
                else (kk % _NBUF) * cb

            def e16_body(e16, carry):
                wv = ew_v[pl.ds(kk * cb + e16 * _LANES, _LANES)]
                for l in range(_LANES):
                    bc = lax.gather(
                        wv, jnp.full((_LANES, 1), l, jnp.int32),
                        lax.GatherDimensionNumbers(
                            offset_dims=(), collapsed_slice_dims=(0,),
                            start_index_map=(0,)),
                        slice_sizes=(1,),
                        mode=lax.GatherScatterMode.PROMISE_IN_BOUNDS)
                    r = boff + e16 * _LANES + l
                    for j in range(nj):
                        sl = pl.ds(j * _LANES, _LANES)
                        rows_v[r, sl] = rows_v[r, sl] * bc
                return carry

            lax.fori_loop(0, cb // _LANES, e16_body, 0)

        def step(kk, first, last):
            if not last:
                if not first:
                    wait_scatter(kk - 2)
                issue_gather(kk + 1)
            wait_gather(kk)
            issue_scatter(kk)

        def group_body(g, carry):
            gi = w * n_groups + g
            pltpu.sync_copy(src_hbm.at[gi], src_v)
            pltpu.sync_copy(dst_hbm.at[gi], dst_v)
            pltpu.sync_copy(ew_hbm.at[gi], ew_v)
            issue_gather(0)
            step(0, True, False)
            step(1, True, False)

            def steady(kk, carry2):
                step(kk, False, False)
                return carry2

            lax.fori_loop(2, _GRP - 1, steady, 0)
            step(_GRP - 1, False, True)
            wait_scatter(_GRP - 3)
            wait_scatter(_GRP - 2)
            wait_scatter(_GRP - 1)
            return carry

        lax.fori_loop(0, n_groups, group_body, 0)
        plsc.subcore_barrier()
        pltpu.sync_copy(
            acc_sh.at[pl.ds(s * rows_per_sub, rows_per_sub)],
            out_hbm.at[pl.ds(c * np_ + s * rows_per_sub, rows_per_sub)])

    return seg_kernel(h, src_g, dst_g, ew_g, zeros_nd)


def _tc_layer(seg2, h, w_rel, b_rel, w_root, relu):
    n, d = h.shape
    bn = 1000
    grid = n // bn

    def body(s_ref, h_ref, wr_ref, br_ref, wo_ref, o_ref):
        aggr = s_ref[0] + s_ref[1]
        r = lax.dot_general(aggr, wr_ref[...], (((1,), (1,)), ((), ())),
                            preferred_element_type=jnp.float32)
        r = r + br_ref[...]
        r = r + lax.dot_general(h_ref[...], wo_ref[...], (((1,), (1,)), ((), ())),
                                preferred_element_type=jnp.float32)
        if relu:
            r = jnp.maximum(r, 0.0)
        o_ref[...] = r

    return pl.pallas_call(
        body,
        grid=(grid,),
        in_specs=[
            pl.BlockSpec((2, bn, d), lambda i: (0, i, 0)),
            pl.BlockSpec((bn, d), lambda i: (i, 0)),
            pl.BlockSpec((d, d), lambda i: (0, 0)),
            pl.BlockSpec((1, d), lambda i: (0, 0)),
            pl.BlockSpec((d, d), lambda i: (0, 0)),
        ],
        out_specs=pl.BlockSpec((bn, d), lambda i: (i, 0)),
        out_shape=jax.ShapeDtypeStruct((n, d), jnp.float32),
    )(seg2, h, w_rel, b_rel.reshape(1, d), w_root)


def kernel(x, edge_index, edge_weight,
           W_rel0, b_rel0, W_root0,
           W_rel1, b_rel1, W_root1,
           W_rel2, b_rel2, W_root2,
           W_rel3, b_rel3, W_root3):
    n, d = x.shape
    e = edge_weight.shape[0]
    k_chunks = -(-(-(-e // (_NW * _CB))) // _GRP) * _GRP
    e_pad = _NW * k_chunks * _CB
    pad = e_pad - e

    n_groups = k_chunks // _GRP
    src = jnp.concatenate([edge_index[0], jnp.zeros((pad,), jnp.int32)])
    dst = jnp.concatenate([edge_index[1], jnp.zeros((pad,), jnp.int32)])
    ew = jnp.concatenate([edge_weight, jnp.zeros((pad,), jnp.float32)])
    src_g = src.reshape(_NW * n_groups, _GRP, _CB)
    dst_g = dst.reshape(_NW * n_groups, _GRP, _CB)
    ew_g = ew.reshape(_NW * n_groups, _GRP * _CB)
    np_ = -(-n // (_NS * 8)) * (_NS * 8)
    zeros_nd = jnp.zeros((np_, d), jnp.float32)

    params = [
        (W_rel0, b_rel0, W_root0),
        (W_rel1, b_rel1, W_root1),
        (W_rel2, b_rel2, W_root2),
        (W_rel3, b_rel3, W_root3),
    ]
    h = x
    for l in range(4):
        w_rel, b_rel, w_root = params[l]
        seg2 = _sc_segsum(h, src_g, dst_g, ew_g, zeros_nd)
        seg2 = seg2.reshape(2, np_, d)
        h = _tc_layer(seg2, h, w_rel, b_rel, w_root, relu=(l < 3))
    return h


# speedup vs baseline: 1.1501x; 1.1501x over previous
import functools

import jax
import jax.numpy as jnp
from jax import lax
from jax.experimental import pallas as pl
from jax.experimental.pallas import tpu as pltpu
from jax.experimental.pallas import tpu_sc as plsc

_NC = 2
_NS = 16
_LANES = 16
_NW = _NC * _NS
_CB = 64
_GRP = 16
_NBUF = 2


def _sc_segsum(h, src_g, dst_g, ew_g, zeros_nd):
    n, d = h.shape
    np_ = zeros_nd.shape[0]
    nwg, grp, cb = src_g.shape
    n_groups = nwg // _NW
    k_chunks = n_groups * grp
    rows_per_sub = np_ // _NS
    nj = d // _LANES

    mesh = plsc.VectorSubcoreMesh(
        core_axis_name="c", subcore_axis_name="s",
        num_cores=_NC, num_subcores=_NS)

    @functools.partial(
        pl.kernel, mesh=mesh,
        out_type=jax.ShapeDtypeStruct((_NC * np_, d), jnp.float32),
        scratch_types=[
            pltpu.VMEM((_GRP, cb), jnp.int32),
            pltpu.VMEM((_GRP, cb), jnp.int32),
            pltpu.VMEM((_GRP * cb,), jnp.float32),
            pltpu.VMEM((_NBUF * cb, d), jnp.float32),
            pltpu.VMEM_SHARED((np_, d), jnp.float32),
            pltpu.SemaphoreType.DMA,
            pltpu.SemaphoreType.DMA,
        ])
    def seg_kernel(h_hbm, src_hbm, dst_hbm, ew_hbm, z_hbm, out_hbm,
                   src_v, dst_v, ew_v, rows_v, acc_sh, gsem, ssem):
        c = lax.axis_index("c")
        s = lax.axis_index("s")
        w = s * _NC + c
        pltpu.sync_copy(z_hbm.at[pl.ds(s * rows_per_sub, rows_per_sub)],
                        acc_sh.at[pl.ds(s * rows_per_sub, rows_per_sub)])
        plsc.subcore_barrier()

        def rows_at(kk):
            boff = lax.rem(kk, _NBUF) * cb if not isinstance(kk, int) \
                else (kk % _NBUF) * cb
            return rows_v.at[pl.ds(boff, cb)]

        def issue_gather(kk):
            pltpu.async_copy(h_hbm.at[src_v.at[kk]], rows_at(kk), gsem)

        def wait_gather(kk):
            pltpu.make_async_copy(h_hbm.at[src_v.at[kk]], rows_at(kk),
                                  gsem).wait()

        def issue_scatter(kk):
            pltpu.async_copy(rows_at(kk), acc_sh.at[dst_v.at[kk]], ssem,
                             add=True)

        def wait_scatter(kk):
            pltpu.make_async_copy(rows_at(kk), acc_sh.at[dst_v.at[kk]],
                                  ssem).wait()

        def scale(kk):
            boff = lax.rem(kk, _NBUF) * cb if not isinstance(kk, int) \
                else (kk % _NBUF) * cb

            def e16_body(e16, carry):
                wv = ew_v[pl.ds(kk * cb + e16 * _LANES, _LANES)]
                for l in range(_LANES):
                    bc = lax.gather(
                        wv, jnp.full((_LANES, 1), l, jnp.int32),
                        lax.GatherDimensionNumbers(
                            offset_dims=(), collapsed_slice_dims=(0,),
                            start_index_map=(0,)),
                        slice_sizes=(1,),
                        mode=lax.GatherScatterMode.PROMISE_IN_BOUNDS)
                    r = boff + e16 * _LANES + l
                    for j in range(nj):
                        sl = pl.ds(j * _LANES, _LANES)
                        rows_v[r, sl] = rows_v[r, sl] * bc
                return carry

            lax.fori_loop(0, cb // _LANES, e16_body, 0)

        def step(kk, first, last):
            if not last:
                if not first:
                    wait_scatter(kk - 1)
                issue_gather(kk + 1)
            wait_gather(kk)
            scale(kk)
            issue_scatter(kk)

        def group_body(g, carry):
            gi = w * n_groups + g
            pltpu.sync_copy(src_hbm.at[gi], src_v)
            pltpu.sync_copy(dst_hbm.at[gi], dst_v)
            pltpu.sync_copy(ew_hbm.at[gi], ew_v)
            issue_gather(0)
            step(0, True, False)

            def steady(kk, carry2):
                step(kk, False, False)
                return carry2

            lax.fori_loop(1, _GRP - 1, steady, 0)
            step(_GRP - 1, False, True)
            wait_scatter(_GRP - 2)
            wait_scatter(_GRP - 1)
            return carry

        lax.fori_loop(0, n_groups, group_body, 0)
        plsc.subcore_barrier()
        pltpu.sync_copy(
            acc_sh.at[pl.ds(s * rows_per_sub, rows_per_sub)],
            out_hbm.at[pl.ds(c * np_ + s * rows_per_sub, rows_per_sub)])

    return seg_kernel(h, src_g, dst_g, ew_g, zeros_nd)


def _tc_layer(seg2, h, w_rel, b_rel, w_root, relu):
    n, d = h.shape
    bn = 1000
    grid = n // bn

    def body(s_ref, h_ref, wr_ref, br_ref, wo_ref, o_ref):
        aggr = s_ref[0] + s_ref[1]
        r = lax.dot_general(aggr, wr_ref[...], (((1,), (1,)), ((), ())),
                            preferred_element_type=jnp.float32)
        r = r + br_ref[...]
        r = r + lax.dot_general(h_ref[...], wo_ref[...], (((1,), (1,)), ((), ())),
                                preferred_element_type=jnp.float32)
        if relu:
            r = jnp.maximum(r, 0.0)
        o_ref[...] = r

    return pl.pallas_call(
        body,
        grid=(grid,),
        in_specs=[
            pl.BlockSpec((2, bn, d), lambda i: (0, i, 0)),
            pl.BlockSpec((bn, d), lambda i: (i, 0)),
            pl.BlockSpec((d, d), lambda i: (0, 0)),
            pl.BlockSpec((1, d), lambda i: (0, 0)),
            pl.BlockSpec((d, d), lambda i: (0, 0)),
        ],
        out_specs=pl.BlockSpec((bn, d), lambda i: (i, 0)),
        out_shape=jax.ShapeDtypeStruct((n, d), jnp.float32),
    )(seg2, h, w_rel, b_rel.reshape(1, d), w_root)


def kernel(x, edge_index, edge_weight,
           W_rel0, b_rel0, W_root0,
           W_rel1, b_rel1, W_root1,
           W_rel2, b_rel2, W_root2,
           W_rel3, b_rel3, W_root3):
    n, d = x.shape
    e = edge_weight.shape[0]
    k_chunks = -(-(-(-e // (_NW * _CB))) // _GRP) * _GRP
    e_pad = _NW * k_chunks * _CB
    pad = e_pad - e

    n_groups = k_chunks // _GRP
    src = jnp.concatenate([edge_index[0], jnp.zeros((pad,), jnp.int32)])
    dst = jnp.concatenate([edge_index[1], jnp.zeros((pad,), jnp.int32)])
    ew = jnp.concatenate([edge_weight, jnp.zeros((pad,), jnp.float32)])
    src_g = src.reshape(_NW * n_groups, _GRP, _CB)
    dst_g = dst.reshape(_NW * n_groups, _GRP, _CB)
    ew_g = ew.reshape(_NW * n_groups, _GRP * _CB)
    np_ = -(-n // (_NS * 8)) * (_NS * 8)
    zeros_nd = jnp.zeros((np_, d), jnp.float32)

    params = [
        (W_rel0, b_rel0, W_root0),
        (W_rel1, b_rel1, W_root1),
        (W_rel2, b_rel2, W_root2),
        (W_rel3, b_rel3, W_root3),
    ]
    h = x
    for l in range(4):
        w_rel, b_rel, w_root = params[l]
        seg2 = _sc_segsum(h, src_g, dst_g, ew_g, zeros_nd)
        seg2 = seg2.reshape(2, np_, d)
        h = _tc_layer(seg2, h, w_rel, b_rel, w_root, relu=(l < 3))
    return h


# speedup vs baseline: 1.4485x; 1.2594x over previous
import functools

import jax
import jax.numpy as jnp
from jax import lax
from jax.experimental import pallas as pl
from jax.experimental.pallas import tpu as pltpu
from jax.experimental.pallas import tpu_sc as plsc

_NC = 2
_NS = 16
_LANES = 16
_NW = _NC * _NS
_CB = 64
_GRP = 16
_NBUF = 2


def _sc_segsum(h, src_g, dst_g, ew_g, zeros_nd):
    n, d = h.shape
    np_ = zeros_nd.shape[0]
    nwg, grp, cb = src_g.shape
    n_groups = nwg // _NW
    k_chunks = n_groups * grp
    rows_per_sub = np_ // _NS
    nj = d // _LANES

    mesh = plsc.VectorSubcoreMesh(
        core_axis_name="c", subcore_axis_name="s",
        num_cores=_NC, num_subcores=_NS)

    @functools.partial(
        pl.kernel, mesh=mesh,
        out_type=jax.ShapeDtypeStruct((_NC * np_, d), jnp.float32),
        scratch_types=[
            pltpu.VMEM((_GRP, cb), jnp.int32),
            pltpu.VMEM((_GRP, cb), jnp.int32),
            pltpu.VMEM((_GRP * cb,), jnp.float32),
            pltpu.VMEM((_NBUF * cb, d), jnp.float32),
            pltpu.VMEM_SHARED((np_, d), jnp.float32),
            pltpu.SemaphoreType.DMA,
            pltpu.SemaphoreType.DMA,
        ])
    def seg_kernel(h_hbm, src_hbm, dst_hbm, ew_hbm, z_hbm, out_hbm,
                   src_v, dst_v, ew_v, rows_v, acc_sh, gsem, ssem):
        c = lax.axis_index("c")
        s = lax.axis_index("s")
        w = s * _NC + c
        pltpu.sync_copy(z_hbm.at[pl.ds(s * rows_per_sub, rows_per_sub)],
                        acc_sh.at[pl.ds(s * rows_per_sub, rows_per_sub)])
        plsc.subcore_barrier()

        def rows_at(kk):
            boff = lax.rem(kk, _NBUF) * cb if not isinstance(kk, int) \
                else (kk % _NBUF) * cb
            return rows_v.at[pl.ds(boff, cb)]

        def issue_gather(kk):
            pltpu.async_copy(h_hbm.at[src_v.at[kk]], rows_at(kk), gsem)

        def wait_gather(kk):
            pltpu.make_async_copy(h_hbm.at[src_v.at[kk]], rows_at(kk),
                                  gsem).wait()

        def issue_scatter(kk):
            pltpu.async_copy(rows_at(kk), acc_sh.at[dst_v.at[kk]], ssem,
                             add=True)

        def wait_scatter(kk):
            pltpu.make_async_copy(rows_at(kk), acc_sh.at[dst_v.at[kk]],
                                  ssem).wait()

        def scale(kk):
            boff = lax.rem(kk, _NBUF) * cb if not isinstance(kk, int) \
                else (kk % _NBUF) * cb

            def e16_body(e16, carry):
                wv = ew_v[pl.ds(kk * cb + e16 * _LANES, _LANES)]
                for l in range(_LANES):
                    bc = lax.gather(
                        wv, jnp.full((_LANES, 1), l, jnp.int32),
                        lax.GatherDimensionNumbers(
                            offset_dims=(), collapsed_slice_dims=(0,),
                            start_index_map=(0,)),
                        slice_sizes=(1,),
                        mode=lax.GatherScatterMode.PROMISE_IN_BOUNDS)
                    r = boff + e16 * _LANES + l
                    for j in range(nj):
                        sl = pl.ds(j * _LANES, _LANES)
                        rows_v[r, sl] = rows_v[r, sl] * bc
                return carry

            lax.fori_loop(0, cb // _LANES, e16_body, 0)

        def step(kk, first, last):
            if not last:
                if not first:
                    wait_scatter(kk - 1)
                issue_gather(kk + 1)
            wait_gather(kk)
            scale(kk)
            issue_scatter(kk)

        def group_body(g, carry):
            gi = w * n_groups + g
            pltpu.sync_copy(src_hbm.at[gi], src_v)
            pltpu.sync_copy(dst_hbm.at[gi], dst_v)
            pltpu.sync_copy(ew_hbm.at[gi], ew_v)
            issue_gather(0)
            step(0, True, False)

            def steady(kk, carry2):
                step(kk, False, False)
                return carry2

            lax.fori_loop(1, _GRP - 1, steady, 0)
            step(_GRP - 1, False, True)
            wait_scatter(_GRP - 2)
            wait_scatter(_GRP - 1)
            return carry

        lax.fori_loop(0, n_groups, group_body, 0)
        plsc.subcore_barrier()
        pltpu.sync_copy(
            acc_sh.at[pl.ds(s * rows_per_sub, rows_per_sub)],
            out_hbm.at[pl.ds(c * np_ + s * rows_per_sub, rows_per_sub)])

    return seg_kernel(h, src_g, dst_g, ew_g, zeros_nd)


def _tc_layer(seg2, h, w_rel, b_rel, w_root, relu):
    n, d = h.shape
    bn = 1000
    grid = n // bn

    def body(s_ref, h_ref, wr_ref, br_ref, wo_ref, o_ref):
        aggr = s_ref[0] + s_ref[1]
        r = lax.dot_general(aggr, wr_ref[...], (((1,), (1,)), ((), ())),
                            preferred_element_type=jnp.float32)
        r = r + br_ref[...]
        r = r + lax.dot_general(h_ref[...], wo_ref[...], (((1,), (1,)), ((), ())),
                                preferred_element_type=jnp.float32)
        if relu:
            r = jnp.maximum(r, 0.0)
        o_ref[...] = r

    return pl.pallas_call(
        body,
        grid=(grid,),
        in_specs=[
            pl.BlockSpec((2, bn, d), lambda i: (0, i, 0)),
            pl.BlockSpec((bn, d), lambda i: (i, 0)),
            pl.BlockSpec((d, d), lambda i: (0, 0)),
            pl.BlockSpec((1, d), lambda i: (0, 0)),
            pl.BlockSpec((d, d), lambda i: (0, 0)),
        ],
        out_specs=pl.BlockSpec((bn, d), lambda i: (i, 0)),
        out_shape=jax.ShapeDtypeStruct((n, d), jnp.float32),
    )(seg2, h, w_rel, b_rel.reshape(1, d), w_root)


def kernel(x, edge_index, edge_weight,
           W_rel0, b_rel0, W_root0,
           W_rel1, b_rel1, W_root1,
           W_rel2, b_rel2, W_root2,
           W_rel3, b_rel3, W_root3):
    n, d = x.shape
    e = edge_weight.shape[0]
    k_chunks = -(-(-(-e // (_NW * _CB))) // _GRP) * _GRP
    e_pad = _NW * k_chunks * _CB
    pad = e_pad - e

    n_groups = k_chunks // _GRP
    pad_idx = (jnp.arange(pad, dtype=jnp.int32) * 13) % n
    src = jnp.concatenate([edge_index[0], pad_idx])
    dst = jnp.concatenate([edge_index[1], pad_idx])
    ew = jnp.concatenate([edge_weight, jnp.zeros((pad,), jnp.float32)])
    src_g = src.reshape(_NW * n_groups, _GRP, _CB)
    dst_g = dst.reshape(_NW * n_groups, _GRP, _CB)
    ew_g = ew.reshape(_NW * n_groups, _GRP * _CB)
    np_ = -(-n // (_NS * 8)) * (_NS * 8)
    zeros_nd = jnp.zeros((np_, d), jnp.float32)

    params = [
        (W_rel0, b_rel0, W_root0),
        (W_rel1, b_rel1, W_root1),
        (W_rel2, b_rel2, W_root2),
        (W_rel3, b_rel3, W_root3),
    ]
    h = x
    for l in range(4):
        w_rel, b_rel, w_root = params[l]
        seg2 = _sc_segsum(h, src_g, dst_g, ew_g, zeros_nd)
        seg2 = seg2.reshape(2, np_, d)
        h = _tc_layer(seg2, h, w_rel, b_rel, w_root, relu=(l < 3))
    return h


# speedup vs baseline: 1.4588x; 1.0071x over previous
import functools

import jax
import jax.numpy as jnp
from jax import lax
from jax.experimental import pallas as pl
from jax.experimental.pallas import tpu as pltpu
from jax.experimental.pallas import tpu_sc as plsc

_NC = 2
_NS = 16
_LANES = 16
_NW = _NC * _NS
_CB = 64
_GRP = 16
_NBUF = 2


def _sc_segsum(h, src_g, dst_g, ew_g, np_):
    n, d = h.shape
    nwg, grp, cb = src_g.shape
    n_groups = nwg // _NW
    k_chunks = n_groups * grp
    rows_per_sub = np_ // _NS
    nj = d // _LANES

    mesh = plsc.VectorSubcoreMesh(
        core_axis_name="c", subcore_axis_name="s",
        num_cores=_NC, num_subcores=_NS)

    @functools.partial(
        pl.kernel, mesh=mesh,
        out_type=jax.ShapeDtypeStruct((_NC * np_, d), jnp.float32),
        scratch_types=[
            pltpu.VMEM((_GRP, cb), jnp.int32),
            pltpu.VMEM((_GRP, cb), jnp.int32),
            pltpu.VMEM((_GRP * cb,), jnp.float32),
            pltpu.VMEM((_NBUF * cb, d), jnp.float32),
            pltpu.VMEM_SHARED((np_, d), jnp.float32),
            pltpu.SemaphoreType.DMA,
            pltpu.SemaphoreType.DMA,
        ])
    def seg_kernel(h_hbm, src_hbm, dst_hbm, ew_hbm, out_hbm,
                   src_v, dst_v, ew_v, rows_v, acc_sh, gsem, ssem):
        c = lax.axis_index("c")
        s = lax.axis_index("s")
        w = s * _NC + c
        zv = jnp.zeros((_LANES,), jnp.float32)

        def zfill(i, carry):
            for j in range(nj):
                rows_v[i, pl.ds(j * _LANES, _LANES)] = zv
            return carry

        zrows = _NBUF * cb
        lax.fori_loop(0, zrows, zfill, 0)
        for t in range(rows_per_sub // zrows):
            pltpu.sync_copy(
                rows_v.at[pl.ds(0, zrows)],
                acc_sh.at[pl.ds(s * rows_per_sub + t * zrows, zrows)])
        rem = rows_per_sub % zrows
        if rem:
            pltpu.sync_copy(
                rows_v.at[pl.ds(0, rem)],
                acc_sh.at[pl.ds(s * rows_per_sub
                                + (rows_per_sub // zrows) * zrows, rem)])
        plsc.subcore_barrier()

        def rows_at(kk):
            boff = lax.rem(kk, _NBUF) * cb if not isinstance(kk, int) \
                else (kk % _NBUF) * cb
            return rows_v.at[pl.ds(boff, cb)]

        def issue_gather(kk):
            pltpu.async_copy(h_hbm.at[src_v.at[kk]], rows_at(kk), gsem)

        def wait_gather(kk):
            pltpu.make_async_copy(h_hbm.at[src_v.at[kk]], rows_at(kk),
                                  gsem).wait()

        def issue_scatter(kk):
            pltpu.async_copy(rows_at(kk), acc_sh.at[dst_v.at[kk]], ssem,
                             add=True)

        def wait_scatter(kk):
            pltpu.make_async_copy(rows_at(kk), acc_sh.at[dst_v.at[kk]],
                                  ssem).wait()

        def scale(kk):
            boff = lax.rem(kk, _NBUF) * cb if not isinstance(kk, int) \
                else (kk % _NBUF) * cb

            def e16_body(e16, carry):
                wv = ew_v[pl.ds(kk * cb + e16 * _LANES, _LANES)]
                for l in range(_LANES):
                    bc = lax.gather(
                        wv, jnp.full((_LANES, 1), l, jnp.int32),
                        lax.GatherDimensionNumbers(
                            offset_dims=(), collapsed_slice_dims=(0,),
                            start_index_map=(0,)),
                        slice_sizes=(1,),
                        mode=lax.GatherScatterMode.PROMISE_IN_BOUNDS)
                    r = boff + e16 * _LANES + l
                    for j in range(nj):
                        sl = pl.ds(j * _LANES, _LANES)
                        rows_v[r, sl] = rows_v[r, sl] * bc
                return carry

            lax.fori_loop(0, cb // _LANES, e16_body, 0)

        def step(kk, first, last):
            if not last:
                if not first:
                    wait_scatter(kk - 1)
                issue_gather(kk + 1)
            wait_gather(kk)
            scale(kk)
            issue_scatter(kk)

        def group_body(g, carry):
            gi = w * n_groups + g
            pltpu.sync_copy(src_hbm.at[gi], src_v)
            pltpu.sync_copy(dst_hbm.at[gi], dst_v)
            pltpu.sync_copy(ew_hbm.at[gi], ew_v)
            issue_gather(0)
            step(0, True, False)

            def steady(kk, carry2):
                step(kk, False, False)
                return carry2

            lax.fori_loop(1, _GRP - 1, steady, 0)
            step(_GRP - 1, False, True)
            wait_scatter(_GRP - 2)
            wait_scatter(_GRP - 1)
            return carry

        lax.fori_loop(0, n_groups, group_body, 0)
        plsc.subcore_barrier()
        pltpu.sync_copy(
            acc_sh.at[pl.ds(s * rows_per_sub, rows_per_sub)],
            out_hbm.at[pl.ds(c * np_ + s * rows_per_sub, rows_per_sub)])

    return seg_kernel(h, src_g, dst_g, ew_g)


def _tc_layer(seg2, h, w_rel, b_rel, w_root, relu):
    n, d = h.shape
    bn = 1000
    grid = n // bn

    def body(s_ref, h_ref, wr_ref, br_ref, wo_ref, o_ref):
        aggr = s_ref[0] + s_ref[1]
        r = lax.dot_general(aggr, wr_ref[...], (((1,), (1,)), ((), ())),
                            preferred_element_type=jnp.float32)
        r = r + br_ref[...]
        r = r + lax.dot_general(h_ref[...], wo_ref[...], (((1,), (1,)), ((), ())),
                                preferred_element_type=jnp.float32)
        if relu:
            r = jnp.maximum(r, 0.0)
        o_ref[...] = r

    return pl.pallas_call(
        body,
        grid=(grid,),
        in_specs=[
            pl.BlockSpec((2, bn, d), lambda i: (0, i, 0)),
            pl.BlockSpec((bn, d), lambda i: (i, 0)),
            pl.BlockSpec((d, d), lambda i: (0, 0)),
            pl.BlockSpec((1, d), lambda i: (0, 0)),
            pl.BlockSpec((d, d), lambda i: (0, 0)),
        ],
        out_specs=pl.BlockSpec((bn, d), lambda i: (i, 0)),
        out_shape=jax.ShapeDtypeStruct((n, d), jnp.float32),
    )(seg2, h, w_rel, b_rel.reshape(1, d), w_root)


def kernel(x, edge_index, edge_weight,
           W_rel0, b_rel0, W_root0,
           W_rel1, b_rel1, W_root1,
           W_rel2, b_rel2, W_root2,
           W_rel3, b_rel3, W_root3):
    n, d = x.shape
    e = edge_weight.shape[0]
    k_chunks = -(-(-(-e // (_NW * _CB))) // _GRP) * _GRP
    e_pad = _NW * k_chunks * _CB
    pad = e_pad - e

    n_groups = k_chunks // _GRP
    pad_idx = (jnp.arange(pad, dtype=jnp.int32) * 13) % n
    src = jnp.concatenate([edge_index[0], pad_idx])
    dst = jnp.concatenate([edge_index[1], pad_idx])
    ew = jnp.concatenate([edge_weight, jnp.zeros((pad,), jnp.float32)])
    src_g = src.reshape(_NW * n_groups, _GRP, _CB)
    dst_g = dst.reshape(_NW * n_groups, _GRP, _CB)
    ew_g = ew.reshape(_NW * n_groups, _GRP * _CB)
    np_ = -(-n // (_NS * 8)) * (_NS * 8)

    params = [
        (W_rel0, b_rel0, W_root0),
        (W_rel1, b_rel1, W_root1),
        (W_rel2, b_rel2, W_root2),
        (W_rel3, b_rel3, W_root3),
    ]
    h = x
    for l in range(4):
        w_rel, b_rel, w_root = params[l]
        seg2 = _sc_segsum(h, src_g, dst_g, ew_g, np_)
        seg2 = seg2.reshape(2, np_, d)
        h = _tc_layer(seg2, h, w_rel, b_rel, w_root, relu=(l < 3))
    return h
